# baseline (speedup 1.0000x reference)
import functools

import jax
import jax.numpy as jnp
from jax import lax
from jax.experimental import pallas as pl
from jax.experimental.pallas import tpu as pltpu
from jax.experimental.pallas import tpu_sc as plsc

N = 10000
E = 320000
D = 128
R = 16
LORA_SCALE = 32.0 / 16.0

NC = 2
NS = 16
NW = NC * NS
BS = 128
NB = 80
EPW = NB * BS
EPAD = NW * EPW
NPAD = 10240
RPS = NPAD // NS

_MESH = plsc.VectorSubcoreMesh(
    core_axis_name="c", subcore_axis_name="s", num_cores=NC, num_subcores=NS
)


@functools.partial(
    pl.kernel,
    out_type=jax.ShapeDtypeStruct((NC, NPAD), jnp.float32),
    mesh=_MESH,
    scratch_types=[
        pltpu.VMEM((NB, BS), jnp.int32),
        pltpu.VMEM((BS,), jnp.float32),
        pltpu.VMEM_SHARED((NPAD,), jnp.float32),
        pltpu.SemaphoreType.DMA,
    ],
)
def _deg_kernel(cols_hbm, zeros_hbm, ones_hbm, out_hbm, cidx_v, ones_v, deg_sh, dsem):
    c = lax.axis_index("c")
    s = lax.axis_index("s")
    pltpu.sync_copy(zeros_hbm, deg_sh.at[pl.ds(s * RPS, RPS)])
    pltpu.sync_copy(ones_hbm, ones_v)
    pltpu.sync_copy(cols_hbm.at[c, s], cidx_v)
    plsc.subcore_barrier()

    def body(j, carry):
        pltpu.async_copy(ones_v, deg_sh.at[cidx_v.at[j]], dsem, add=True)
        return carry

    lax.fori_loop(0, NB, body, 0)

    def drain(j, carry):
        pltpu.make_async_copy(ones_v, deg_sh.at[cidx_v.at[0]], dsem).wait()
        return carry

    lax.fori_loop(0, NB, drain, 0)
    plsc.subcore_barrier()
    pltpu.sync_copy(
        deg_sh.at[pl.ds(s * RPS, RPS)], out_hbm.at[c, pl.ds(s * RPS, RPS)]
    )


SBS = 128
SNB = EPW // SBS // 2
NCHUNK = 2
NBH = SNB // NCHUNK


@functools.partial(
    pl.kernel,
    out_type=jax.ShapeDtypeStruct((NC, NPAD, D), jnp.float32),
    mesh=_MESH,
    scratch_types=[
        pltpu.VMEM((NBH, SBS), jnp.int32),
        pltpu.VMEM((NBH, SBS), jnp.int32),
        pltpu.VMEM((2, SBS, D), jnp.float32),
        pltpu.VMEM_SHARED((NPAD, D), jnp.float32),
        pltpu.SemaphoreType.DMA,
        pltpu.SemaphoreType.DMA,
    ],
)
def _scatter_kernel(
    g_hbm, rows_hbm, cols_hbm, zeros_hbm, out_hbm,
    ridx_v, cidx_v, buf_v, acc_sh, sem0, sem1,
):
    c = lax.axis_index("c")
    s = lax.axis_index("s")
    pltpu.sync_copy(zeros_hbm, acc_sh.at[pl.ds(s * RPS, RPS)])
    plsc.subcore_barrier()

    sems = (sem0, sem1)
    for half in range(NCHUNK):
        pltpu.sync_copy(rows_hbm.at[c, s, half], ridx_v)
        pltpu.sync_copy(cols_hbm.at[c, s, half], cidx_v)
        pltpu.async_copy(g_hbm.at[ridx_v.at[0]], buf_v.at[0], sems[0])
        pltpu.async_copy(g_hbm.at[ridx_v.at[1]], buf_v.at[1], sems[1])

        def body(i, carry):
            for b in range(2):
                j = 2 * i + b
                pltpu.make_async_copy(
                    g_hbm.at[ridx_v.at[0]], buf_v.at[b], sems[b]
                ).wait()
                pltpu.sync_copy(buf_v.at[b], acc_sh.at[cidx_v.at[j]], add=True)

                @pl.when(j + 2 < NBH)
                def _prefetch():
                    pltpu.async_copy(
                        g_hbm.at[ridx_v.at[j + 2]], buf_v.at[b], sems[b]
                    )

            return carry

        lax.fori_loop(0, NBH // 2, body, 0)

    plsc.subcore_barrier()
    pltpu.sync_copy(
        acc_sh.at[pl.ds(s * RPS, RPS)], out_hbm.at[c, pl.ds(s * RPS, RPS)]
    )


def _prescale_body(dge_ref, h_ref, g_ref, a_ref):
    deg = dge_ref[0] + dge_ref[1] + 1.0
    dis = lax.rsqrt(deg)
    g_ref[...] = h_ref[...] * dis
    a_ref[...] = dis / deg


_PRE_BLK = 1000


def _prescale(dge, h):
    grid = N // _PRE_BLK
    return pl.pallas_call(
        _prescale_body,
        grid=(grid,),
        in_specs=[
            pl.BlockSpec((NC, _PRE_BLK, 1), lambda i: (0, i, 0)),
            pl.BlockSpec((_PRE_BLK, D), lambda i: (i, 0)),
        ],
        out_specs=[
            pl.BlockSpec((_PRE_BLK, D), lambda i: (i, 0)),
            pl.BlockSpec((_PRE_BLK, 1), lambda i: (i, 0)),
        ],
        out_shape=[
            jax.ShapeDtypeStruct((N, D), jnp.float32),
            jax.ShapeDtypeStruct((N, 1), jnp.float32),
        ],
    )(dge, h)


_DOT = functools.partial(jnp.dot, preferred_element_type=jnp.float32)


def _dense_body(
    h_ref, sp_ref, sq_ref, g_ref, a_ref, wmt, bmr, amt, bmt, wgt, bgr, agt, bgt,
    gam, bet, o_ref,
):
    h = h_ref[...]
    mo = a_ref[...] * (
        sp_ref[0] + sp_ref[1] + sq_ref[0] + sq_ref[1] + g_ref[...]
    )
    mt = _DOT(mo, wmt[...]) + bmr[...] + LORA_SCALE * _DOT(_DOT(mo, amt[...]), bmt[...])
    gl = _DOT(h, wgt[...]) + bgr[...] + LORA_SCALE * _DOT(_DOT(h, agt[...]), bgt[...])
    y = h + jax.nn.sigmoid(gl) * mt
    mu = jnp.mean(y, axis=-1, keepdims=True)
    yc = y - mu
    var = jnp.mean(yc * yc, axis=-1, keepdims=True)
    o_ref[...] = yc * lax.rsqrt(var + 1e-5) * gam[...] + bet[...]


_DN_BLK = 1000


def _dense(h, sp, sq, g, a_col, wmt, bmr, amt, bmt, wgt, bgr, agt, bgt, gam, bet):
    grid = N // _DN_BLK
    full = lambda i: (0, 0)
    return pl.pallas_call(
        _dense_body,
        grid=(grid,),
        in_specs=[
            pl.BlockSpec((_DN_BLK, D), lambda i: (i, 0)),
            pl.BlockSpec((NC, _DN_BLK, D), lambda i: (0, i, 0)),
            pl.BlockSpec((NC, _DN_BLK, D), lambda i: (0, i, 0)),
            pl.BlockSpec((_DN_BLK, D), lambda i: (i, 0)),
            pl.BlockSpec((_DN_BLK, 1), lambda i: (i, 0)),
            pl.BlockSpec((D, D), full),
            pl.BlockSpec((1, D), full),
            pl.BlockSpec((D, R), full),
            pl.BlockSpec((R, D), full),
            pl.BlockSpec((D, D), full),
            pl.BlockSpec((1, D), full),
            pl.BlockSpec((D, R), full),
            pl.BlockSpec((R, D), full),
            pl.BlockSpec((1, D), full),
            pl.BlockSpec((1, D), full),
        ],
        out_specs=pl.BlockSpec((_DN_BLK, D), lambda i: (i, 0)),
        out_shape=jax.ShapeDtypeStruct((N, D), jnp.float32),
    )(h, sp, sq, g, a_col, wmt, bmr, amt, bmt, wgt, bgr, agt, bgt, gam, bet)


def kernel(hidden_states, edge_index, Wm, bm, Am, Bm, Wg, bg, Ag, Bg, gamma, beta):
    h = hidden_states
    npad = EPAD - E
    pad_rows = (jnp.arange(npad, dtype=jnp.int32) * 37) % N
    pad_cols = N + (jnp.arange(npad, dtype=jnp.int32) % (NPAD - N))
    rows = jnp.concatenate([edge_index[0], pad_rows]).reshape(NC, NS, NB, BS)
    cols = jnp.concatenate([edge_index[1], pad_cols]).reshape(NC, NS, NB, BS)
    rows_s = rows.reshape(NC, NS, 2, NCHUNK, NBH, SBS)
    cols_s = cols.reshape(NC, NS, 2, NCHUNK, NBH, SBS)

    zeros1 = jnp.zeros((RPS,), jnp.float32)
    ones_b = jnp.ones((BS,), jnp.float32)
    degp = _deg_kernel(cols, zeros1, ones_b)

    dge = degp[:, :N, None]
    g, a_col = _prescale(dge, h)

    zeros2 = jnp.zeros((RPS, D), jnp.float32)
    sp = _scatter_kernel(g, rows_s[:, :, 0], cols_s[:, :, 0], zeros2)
    sq = _scatter_kernel(g, rows_s[:, :, 1], cols_s[:, :, 1], zeros2)
    sp = sp[:, :N]
    sq = sq[:, :N]

    out = _dense(
        h, sp, sq, g, a_col,
        Wm.T, bm[None, :], Am.T, Bm.T, Wg.T, bg[None, :], Ag.T, Bg.T,
        gamma[None, :], beta[None, :],
    )
    return out


# speedup vs baseline: 1.1582x; 1.1582x over previous
import functools

import jax
import jax.numpy as jnp
from jax import lax
from jax.experimental import pallas as pl
from jax.experimental.pallas import tpu as pltpu
from jax.experimental.pallas import tpu_sc as plsc

N = 10000
E = 320000
D = 128
R = 16
LORA_SCALE = 32.0 / 16.0

NC = 2
NS = 16
NW = NC * NS
BS = 128
NB = 80
EPW = NB * BS
EPAD = NW * EPW
NPAD = 10240
RPS = NPAD // NS

_MESH = plsc.VectorSubcoreMesh(
    core_axis_name="c", subcore_axis_name="s", num_cores=NC, num_subcores=NS
)


@functools.partial(
    pl.kernel,
    out_type=jax.ShapeDtypeStruct((NC, NPAD), jnp.float32),
    mesh=_MESH,
    scratch_types=[
        pltpu.VMEM((NB, BS), jnp.int32),
        pltpu.VMEM((BS,), jnp.float32),
        pltpu.VMEM_SHARED((NPAD,), jnp.float32),
        pltpu.SemaphoreType.DMA,
    ],
)
def _deg_kernel(cols_hbm, zeros_hbm, ones_hbm, out_hbm, cidx_v, ones_v, deg_sh, dsem):
    c = lax.axis_index("c")
    s = lax.axis_index("s")
    pltpu.sync_copy(zeros_hbm, deg_sh.at[pl.ds(s * RPS, RPS)])
    pltpu.sync_copy(ones_hbm, ones_v)
    pltpu.sync_copy(cols_hbm.at[c, s], cidx_v)
    plsc.subcore_barrier()

    def body(j, carry):
        pltpu.async_copy(ones_v, deg_sh.at[cidx_v.at[j]], dsem, add=True)
        return carry

    lax.fori_loop(0, NB, body, 0)

    def drain(j, carry):
        pltpu.make_async_copy(ones_v, deg_sh.at[cidx_v.at[0]], dsem).wait()
        return carry

    lax.fori_loop(0, NB, drain, 0)
    plsc.subcore_barrier()
    pltpu.sync_copy(
        deg_sh.at[pl.ds(s * RPS, RPS)], out_hbm.at[c, pl.ds(s * RPS, RPS)]
    )


SBS = 128
SNB = EPW // SBS
NCHUNK = 2
NBH = SNB // NCHUNK


@functools.partial(
    pl.kernel,
    out_type=jax.ShapeDtypeStruct((NC, NPAD, D), jnp.float32),
    mesh=_MESH,
    scratch_types=[
        pltpu.VMEM((NBH, SBS), jnp.int32),
        pltpu.VMEM((NBH, SBS), jnp.int32),
        pltpu.VMEM((2, SBS, D), jnp.float32),
        pltpu.VMEM_SHARED((NPAD, D), jnp.float32),
        pltpu.SemaphoreType.DMA,
        pltpu.SemaphoreType.DMA,
    ],
)
def _scatter_kernel(
    g_hbm, rows_hbm, cols_hbm, zeros_hbm, out_hbm,
    ridx_v, cidx_v, buf_v, acc_sh, sem0, sem1,
):
    c = lax.axis_index("c")
    s = lax.axis_index("s")
    pltpu.sync_copy(zeros_hbm, acc_sh.at[pl.ds(s * RPS, RPS)])
    plsc.subcore_barrier()

    sems = (sem0, sem1)
    for half in range(NCHUNK):
        pltpu.sync_copy(rows_hbm.at[c, s, half], ridx_v)
        pltpu.sync_copy(cols_hbm.at[c, s, half], cidx_v)
        pltpu.async_copy(g_hbm.at[ridx_v.at[0]], buf_v.at[0], sems[0])
        pltpu.async_copy(g_hbm.at[ridx_v.at[1]], buf_v.at[1], sems[1])

        def body(i, carry):
            for b in range(2):
                j = 2 * i + b
                pltpu.make_async_copy(
                    g_hbm.at[ridx_v.at[0]], buf_v.at[b], sems[b]
                ).wait()
                pltpu.sync_copy(buf_v.at[b], acc_sh.at[cidx_v.at[j]], add=True)

                @pl.when(j + 2 < NBH)
                def _prefetch():
                    pltpu.async_copy(
                        g_hbm.at[ridx_v.at[j + 2]], buf_v.at[b], sems[b]
                    )

            return carry

        lax.fori_loop(0, NBH // 2, body, 0)

    plsc.subcore_barrier()
    pltpu.sync_copy(
        acc_sh.at[pl.ds(s * RPS, RPS)], out_hbm.at[c, pl.ds(s * RPS, RPS)]
    )


def _prescale_body(dge_ref, h_ref, g_ref, a_ref):
    deg = dge_ref[0] + dge_ref[1] + 1.0
    dis = lax.rsqrt(deg)
    g_ref[...] = h_ref[...] * dis
    a_ref[...] = dis / deg


_PRE_BLK = 1000


def _prescale(dge, h):
    grid = N // _PRE_BLK
    return pl.pallas_call(
        _prescale_body,
        grid=(grid,),
        in_specs=[
            pl.BlockSpec((NC, _PRE_BLK, 1), lambda i: (0, i, 0)),
            pl.BlockSpec((_PRE_BLK, D), lambda i: (i, 0)),
        ],
        out_specs=[
            pl.BlockSpec((_PRE_BLK, D), lambda i: (i, 0)),
            pl.BlockSpec((_PRE_BLK, 1), lambda i: (i, 0)),
        ],
        out_shape=[
            jax.ShapeDtypeStruct((N, D), jnp.float32),
            jax.ShapeDtypeStruct((N, 1), jnp.float32),
        ],
    )(dge, h)


_DOT = functools.partial(jnp.dot, preferred_element_type=jnp.float32)


def _dense_body(
    h_ref, sp_ref, g_ref, a_ref, wmt, bmr, amt, bmt, wgt, bgr, agt, bgt,
    gam, bet, o_ref,
):
    h = h_ref[...]
    mo = a_ref[...] * (sp_ref[0] + sp_ref[1] + g_ref[...])
    mt = _DOT(mo, wmt[...]) + bmr[...] + LORA_SCALE * _DOT(_DOT(mo, amt[...]), bmt[...])
    gl = _DOT(h, wgt[...]) + bgr[...] + LORA_SCALE * _DOT(_DOT(h, agt[...]), bgt[...])
    y = h + jax.nn.sigmoid(gl) * mt
    mu = jnp.mean(y, axis=-1, keepdims=True)
    yc = y - mu
    var = jnp.mean(yc * yc, axis=-1, keepdims=True)
    o_ref[...] = yc * lax.rsqrt(var + 1e-5) * gam[...] + bet[...]


_DN_BLK = 1000


def _dense(h, sp, g, a_col, wmt, bmr, amt, bmt, wgt, bgr, agt, bgt, gam, bet):
    grid = N // _DN_BLK
    full = lambda i: (0, 0)
    return pl.pallas_call(
        _dense_body,
        grid=(grid,),
        in_specs=[
            pl.BlockSpec((_DN_BLK, D), lambda i: (i, 0)),
            pl.BlockSpec((NC, _DN_BLK, D), lambda i: (0, i, 0)),
            pl.BlockSpec((_DN_BLK, D), lambda i: (i, 0)),
            pl.BlockSpec((_DN_BLK, 1), lambda i: (i, 0)),
            pl.BlockSpec((D, D), full),
            pl.BlockSpec((1, D), full),
            pl.BlockSpec((D, R), full),
            pl.BlockSpec((R, D), full),
            pl.BlockSpec((D, D), full),
            pl.BlockSpec((1, D), full),
            pl.BlockSpec((D, R), full),
            pl.BlockSpec((R, D), full),
            pl.BlockSpec((1, D), full),
            pl.BlockSpec((1, D), full),
        ],
        out_specs=pl.BlockSpec((_DN_BLK, D), lambda i: (i, 0)),
        out_shape=jax.ShapeDtypeStruct((N, D), jnp.float32),
    )(h, sp, g, a_col, wmt, bmr, amt, bmt, wgt, bgr, agt, bgt, gam, bet)


def kernel(hidden_states, edge_index, Wm, bm, Am, Bm, Wg, bg, Ag, Bg, gamma, beta):
    h = hidden_states
    npad = EPAD - E
    pad_rows = (jnp.arange(npad, dtype=jnp.int32) * 37) % N
    pad_cols = N + (jnp.arange(npad, dtype=jnp.int32) % (NPAD - N))
    rows = jnp.concatenate([edge_index[0], pad_rows]).reshape(NC, NS, NB, BS)
    cols = jnp.concatenate([edge_index[1], pad_cols]).reshape(NC, NS, NB, BS)
    rows_s = rows.reshape(NC, NS, NCHUNK, NBH, SBS)
    cols_s = cols.reshape(NC, NS, NCHUNK, NBH, SBS)

    zeros1 = jnp.zeros((RPS,), jnp.float32)
    ones_b = jnp.ones((BS,), jnp.float32)
    degp = _deg_kernel(cols, zeros1, ones_b)

    dge = degp[:, :N, None]
    g, a_col = _prescale(dge, h)

    zeros2 = jnp.zeros((RPS, D), jnp.float32)
    sp = _scatter_kernel(g, rows_s, cols_s, zeros2)
    sp = sp[:, :N]

    out = _dense(
        h, sp, g, a_col,
        Wm.T, bm[None, :], Am.T, Bm.T, Wg.T, bg[None, :], Ag.T, Bg.T,
        gamma[None, :], beta[None, :],
    )
    return out
